# Initial kernel scaffold; baseline (speedup 1.0000x reference)
import jax
import jax.numpy as jnp
from jax.experimental import pallas as pl


def kernel(at_no, pos, edge_index, shifts, embed_table, W_lin, b_lin):
    raise NotImplementedError("write your pallas kernel here")


# speedup vs baseline: 2.4204x; 2.4204x over previous
import functools
import math

import jax
import jax.numpy as jnp
from jax import lax
from jax.experimental import pallas as pl
from jax.experimental.pallas import tpu as pltpu
from jax.experimental.pallas import tpu_sc as plsc

N_NODES = 10000
N_EDGES = 160000
NUM_ELEMENTS = 87
EMBED_DIM = 56
NODE_DIM = 128
NUM_BASIS = 20
CUTOFF = 5.0

NW = 32
EPW = 5008
PLANE = NW * EPW
PAD = PLANE - N_EDGES
NCHUNK = EPW // 16

EB = 640
NB = 1000

_SQ3 = math.sqrt(3.0)
_SQ5 = math.sqrt(5.0)
_SQ15 = math.sqrt(15.0)


def _sc_edge_vec_body(pos_hbm, src_hbm, dst_hbm, shx_hbm, shy_hbm, shz_hbm,
                      out_hbm, pos_v, src_v, dst_v, shx_v, shy_v, shz_v,
                      vx_v, vy_v, vz_v):
    wid = lax.axis_index("s") * 2 + lax.axis_index("c")
    base = wid * EPW
    pltpu.sync_copy(pos_hbm, pos_v)
    pltpu.sync_copy(src_hbm.at[pl.ds(base, EPW)], src_v)
    pltpu.sync_copy(dst_hbm.at[pl.ds(base, EPW)], dst_v)
    pltpu.sync_copy(shx_hbm.at[pl.ds(base, EPW)], shx_v)
    pltpu.sync_copy(shy_hbm.at[pl.ds(base, EPW)], shy_v)
    pltpu.sync_copy(shz_hbm.at[pl.ds(base, EPW)], shz_v)

    def body(i, carry):
        off = i * 16
        s3 = src_v[pl.ds(off, 16)] * 3
        d3 = dst_v[pl.ds(off, 16)] * 3
        vx_v[pl.ds(off, 16)] = (plsc.load_gather(pos_v, [s3])
                                - plsc.load_gather(pos_v, [d3])
                                - shx_v[pl.ds(off, 16)])
        vy_v[pl.ds(off, 16)] = (plsc.load_gather(pos_v, [s3 + 1])
                                - plsc.load_gather(pos_v, [d3 + 1])
                                - shy_v[pl.ds(off, 16)])
        vz_v[pl.ds(off, 16)] = (plsc.load_gather(pos_v, [s3 + 2])
                                - plsc.load_gather(pos_v, [d3 + 2])
                                - shz_v[pl.ds(off, 16)])
        return carry

    lax.fori_loop(0, NCHUNK, body, 0)
    pltpu.sync_copy(vx_v, out_hbm.at[pl.ds(base, EPW)])
    pltpu.sync_copy(vy_v, out_hbm.at[pl.ds(PLANE + base, EPW)])
    pltpu.sync_copy(vz_v, out_hbm.at[pl.ds(2 * PLANE + base, EPW)])


@functools.lru_cache(maxsize=None)
def _sc_edge_vec():
    return pl.kernel(
        _sc_edge_vec_body,
        mesh=plsc.VectorSubcoreMesh(core_axis_name="c", subcore_axis_name="s"),
        compiler_params=pltpu.CompilerParams(needs_layout_passes=False),
        out_type=jax.ShapeDtypeStruct((3 * PLANE,), jnp.float32),
        scratch_types=[
            pltpu.VMEM((3 * N_NODES,), jnp.float32),
            pltpu.VMEM((EPW,), jnp.int32),
            pltpu.VMEM((EPW,), jnp.int32),
            pltpu.VMEM((EPW,), jnp.float32),
            pltpu.VMEM((EPW,), jnp.float32),
            pltpu.VMEM((EPW,), jnp.float32),
            pltpu.VMEM((EPW,), jnp.float32),
            pltpu.VMEM((EPW,), jnp.float32),
            pltpu.VMEM((EPW,), jnp.float32),
        ],
    )


def _tc_edge_body(vec_ref, rbf_ref, fcut_ref, rsh_ref):
    v = vec_ref[...]
    a = v[0:1, :]
    b = v[1:2, :]
    c = v[2:3, :]
    d = jnp.sqrt(a * a + b * b + c * c)
    invd = 1.0 / d
    invd2 = invd * invd

    t = (math.pi / CUTOFF) * d
    st = jnp.sin(t)
    ct = jnp.cos(t)
    two_ct = 2.0 * ct
    rows = [st]
    s_prev, s_cur = jnp.zeros_like(st), st
    for _ in range(NUM_BASIS - 1):
        s_prev, s_cur = s_cur, two_ct * s_cur - s_prev
        rows.append(s_cur)
    sinT = jnp.concatenate(rows, axis=0)
    sinT = sinT * (math.sqrt(2.0 / CUTOFF) * invd)
    eye = (lax.broadcasted_iota(jnp.int32, (NUM_BASIS, NUM_BASIS), 0)
           == lax.broadcasted_iota(jnp.int32, (NUM_BASIS, NUM_BASIS), 1)
           ).astype(jnp.float32)
    rbf_ref[...] = lax.dot_general(sinT, eye, (((0,), (0,)), ((), ())),
                                   preferred_element_type=jnp.float32)

    fc = 0.5 * (ct + 1.0) * (d < CUTOFF).astype(jnp.float32)
    fcut_ref[...] = lax.dot_general(fc, jnp.ones((1, 1), jnp.float32),
                                    (((0,), (0,)), ((), ())),
                                    preferred_element_type=jnp.float32)

    g = _SQ3 * invd
    aT = jnp.concatenate([
        jnp.ones_like(d),
        g * b,
        g * c,
        g * a,
        (_SQ15 * invd2) * (b * a),
        (_SQ15 * invd2) * (b * c),
        (_SQ5 * invd2) * (c * c - 0.5 * (b * b + a * a)),
        (_SQ15 * invd2) * (c * a),
        (0.5 * _SQ15 * invd2) * (a * a - b * b),
    ], axis=0)

    col = lax.broadcasted_iota(jnp.int32, (9, 480), 1)
    rowsel = jnp.where(col < 128, 0,
                       jnp.where(col < 320, 1 + (col - 128) % 3,
                                 4 + (col - 320) % 5))
    sel = (lax.broadcasted_iota(jnp.int32, (9, 480), 0)
           == rowsel).astype(jnp.float32)
    rsh_ref[...] = lax.dot_general(aT, sel, (((0,), (0,)), ((), ())),
                                   preferred_element_type=jnp.float32)


def _tc_edges(vecT):
    grid = N_EDGES // EB
    return pl.pallas_call(
        _tc_edge_body,
        grid=(grid,),
        in_specs=[pl.BlockSpec((3, EB), lambda i: (0, i))],
        out_specs=[
            pl.BlockSpec((EB, NUM_BASIS), lambda i: (i, 0)),
            pl.BlockSpec((EB, 1), lambda i: (i, 0)),
            pl.BlockSpec((EB, 480), lambda i: (i, 0)),
        ],
        out_shape=[
            jax.ShapeDtypeStruct((N_EDGES, NUM_BASIS), jnp.float32),
            jax.ShapeDtypeStruct((N_EDGES, 1), jnp.float32),
            jax.ShapeDtypeStruct((N_EDGES, 480), jnp.float32),
        ],
    )(vecT)


def _tc_node_body(ids_ref, emb_ref, w_ref, b_ref, out_ref):
    ids = ids_ref[0]
    onehotT = (lax.broadcasted_iota(jnp.int32, (NUM_ELEMENTS, NB), 0)
               == ids).astype(jnp.float32)
    fused = lax.dot_general(emb_ref[...], w_ref[...],
                            (((1,), (1,)), ((), ())),
                            preferred_element_type=jnp.float32)
    out_ref[...] = lax.dot_general(onehotT, fused, (((0,), (0,)), ((), ())),
                                   preferred_element_type=jnp.float32) \
        + b_ref[...]


def _tc_nodes(ids3, embed_table, W_lin, b_row):
    grid = N_NODES // NB
    return pl.pallas_call(
        _tc_node_body,
        grid=(grid,),
        in_specs=[
            pl.BlockSpec((1, 1, NB), lambda i: (i, 0, 0)),
            pl.BlockSpec((NUM_ELEMENTS, EMBED_DIM), lambda i: (0, 0)),
            pl.BlockSpec((NODE_DIM, EMBED_DIM), lambda i: (0, 0)),
            pl.BlockSpec((1, NODE_DIM), lambda i: (0, 0)),
        ],
        out_specs=pl.BlockSpec((NB, NODE_DIM), lambda i: (i, 0)),
        out_shape=jax.ShapeDtypeStruct((N_NODES, NODE_DIM), jnp.float32),
    )(ids3, embed_table, W_lin, b_row)


def kernel(at_no, pos, edge_index, shifts, embed_table, W_lin, b_lin):
    pos_flat = pos.reshape(-1)
    src = jnp.pad(edge_index[0], (0, PAD))
    dst = jnp.pad(edge_index[1], (0, PAD))
    shT = jnp.pad(shifts.T, ((0, 0), (0, PAD)))

    vec_flat = _sc_edge_vec()(pos_flat, src, dst, shT[0], shT[1], shT[2])
    vecT = vec_flat.reshape(3, PLANE)

    rbf, fcut, rsh = _tc_edges(vecT)
    x_scalar = _tc_nodes(at_no.reshape(N_NODES // NB, 1, NB),
                         embed_table, W_lin, b_lin.reshape(1, NODE_DIM))
    return x_scalar, rbf, fcut, rsh


# speedup vs baseline: 2.8150x; 1.1630x over previous
import functools
import math

import jax
import jax.numpy as jnp
from jax import lax
from jax.experimental import pallas as pl
from jax.experimental.pallas import tpu as pltpu
from jax.experimental.pallas import tpu_sc as plsc

N_NODES = 10000
N_EDGES = 160000
NUM_ELEMENTS = 87
EMBED_DIM = 56
NODE_DIM = 128
NUM_BASIS = 20
CUTOFF = 5.0

NW = 32
EPW = 5008
PLANE = NW * EPW
PAD = PLANE - N_EDGES
NCHUNK = EPW // 16

EB = 3200
NB = 1000

_SQ3 = math.sqrt(3.0)
_SQ5 = math.sqrt(5.0)
_SQ15 = math.sqrt(15.0)


def _sc_edge_vec_body(pos_hbm, src_hbm, dst_hbm, shx_hbm, shy_hbm, shz_hbm,
                      out_hbm, pos_v, src_v, dst_v, shx_v, shy_v, shz_v,
                      vx_v, vy_v, vz_v):
    wid = lax.axis_index("s") * 2 + lax.axis_index("c")
    base = wid * EPW
    pltpu.sync_copy(pos_hbm, pos_v)
    pltpu.sync_copy(src_hbm.at[pl.ds(base, EPW)], src_v)
    pltpu.sync_copy(dst_hbm.at[pl.ds(base, EPW)], dst_v)
    pltpu.sync_copy(shx_hbm.at[pl.ds(base, EPW)], shx_v)
    pltpu.sync_copy(shy_hbm.at[pl.ds(base, EPW)], shy_v)
    pltpu.sync_copy(shz_hbm.at[pl.ds(base, EPW)], shz_v)

    def body(i, carry):
        off = i * 16
        s3 = src_v[pl.ds(off, 16)] * 3
        d3 = dst_v[pl.ds(off, 16)] * 3
        vx_v[pl.ds(off, 16)] = (plsc.load_gather(pos_v, [s3])
                                - plsc.load_gather(pos_v, [d3])
                                - shx_v[pl.ds(off, 16)])
        vy_v[pl.ds(off, 16)] = (plsc.load_gather(pos_v, [s3 + 1])
                                - plsc.load_gather(pos_v, [d3 + 1])
                                - shy_v[pl.ds(off, 16)])
        vz_v[pl.ds(off, 16)] = (plsc.load_gather(pos_v, [s3 + 2])
                                - plsc.load_gather(pos_v, [d3 + 2])
                                - shz_v[pl.ds(off, 16)])
        return carry

    lax.fori_loop(0, NCHUNK, body, 0)
    pltpu.sync_copy(vx_v, out_hbm.at[pl.ds(base, EPW)])
    pltpu.sync_copy(vy_v, out_hbm.at[pl.ds(PLANE + base, EPW)])
    pltpu.sync_copy(vz_v, out_hbm.at[pl.ds(2 * PLANE + base, EPW)])


@functools.lru_cache(maxsize=None)
def _sc_edge_vec():
    return pl.kernel(
        _sc_edge_vec_body,
        mesh=plsc.VectorSubcoreMesh(core_axis_name="c", subcore_axis_name="s"),
        compiler_params=pltpu.CompilerParams(needs_layout_passes=False),
        out_type=jax.ShapeDtypeStruct((3 * PLANE,), jnp.float32),
        scratch_types=[
            pltpu.VMEM((3 * N_NODES,), jnp.float32),
            pltpu.VMEM((EPW,), jnp.int32),
            pltpu.VMEM((EPW,), jnp.int32),
            pltpu.VMEM((EPW,), jnp.float32),
            pltpu.VMEM((EPW,), jnp.float32),
            pltpu.VMEM((EPW,), jnp.float32),
            pltpu.VMEM((EPW,), jnp.float32),
            pltpu.VMEM((EPW,), jnp.float32),
            pltpu.VMEM((EPW,), jnp.float32),
        ],
    )


def _tc_edge_body(vec_ref, rbf_ref, fcut_ref, rsh_ref):
    v = vec_ref[...]
    a = v[0:1, :]
    b = v[1:2, :]
    c = v[2:3, :]
    d = jnp.sqrt(a * a + b * b + c * c)
    invd = 1.0 / d
    invd2 = invd * invd

    t = (math.pi / CUTOFF) * d
    st = jnp.sin(t)
    ct = jnp.cos(t)
    two_ct = 2.0 * ct
    rows = [st]
    s_prev, s_cur = jnp.zeros_like(st), st
    for _ in range(NUM_BASIS - 1):
        s_prev, s_cur = s_cur, two_ct * s_cur - s_prev
        rows.append(s_cur)
    sinT = jnp.concatenate(rows, axis=0)
    sinT = sinT * (math.sqrt(2.0 / CUTOFF) * invd)
    eye = (lax.broadcasted_iota(jnp.int32, (NUM_BASIS, NUM_BASIS), 0)
           == lax.broadcasted_iota(jnp.int32, (NUM_BASIS, NUM_BASIS), 1)
           ).astype(jnp.float32)
    rbf_ref[...] = lax.dot_general(sinT, eye, (((0,), (0,)), ((), ())),
                                   preferred_element_type=jnp.float32)

    fc = 0.5 * (ct + 1.0) * (d < CUTOFF).astype(jnp.float32)
    fcut_ref[...] = lax.dot_general(fc, jnp.ones((1, 1), jnp.float32),
                                    (((0,), (0,)), ((), ())),
                                    preferred_element_type=jnp.float32)

    g = _SQ3 * invd
    aT = jnp.concatenate([
        jnp.ones_like(d),
        g * b,
        g * c,
        g * a,
        (_SQ15 * invd2) * (b * a),
        (_SQ15 * invd2) * (b * c),
        (_SQ5 * invd2) * (c * c - 0.5 * (b * b + a * a)),
        (_SQ15 * invd2) * (c * a),
        (0.5 * _SQ15 * invd2) * (a * a - b * b),
    ], axis=0)

    col = lax.broadcasted_iota(jnp.int32, (9, 480), 1)
    rowsel = jnp.where(col < 128, 0,
                       jnp.where(col < 320, 1 + (col - 128) % 3,
                                 4 + (col - 320) % 5))
    sel = (lax.broadcasted_iota(jnp.int32, (9, 480), 0)
           == rowsel).astype(jnp.float32)
    rsh_ref[...] = lax.dot_general(aT, sel, (((0,), (0,)), ((), ())),
                                   preferred_element_type=jnp.float32)


def _tc_edges(vecT):
    grid = N_EDGES // EB
    return pl.pallas_call(
        _tc_edge_body,
        grid=(grid,),
        in_specs=[pl.BlockSpec((3, EB), lambda i: (0, i))],
        out_specs=[
            pl.BlockSpec((EB, NUM_BASIS), lambda i: (i, 0)),
            pl.BlockSpec((EB, 1), lambda i: (i, 0)),
            pl.BlockSpec((EB, 480), lambda i: (i, 0)),
        ],
        out_shape=[
            jax.ShapeDtypeStruct((N_EDGES, NUM_BASIS), jnp.float32),
            jax.ShapeDtypeStruct((N_EDGES, 1), jnp.float32),
            jax.ShapeDtypeStruct((N_EDGES, 480), jnp.float32),
        ],
    )(vecT)


def _tc_node_body(ids_ref, emb_ref, w_ref, b_ref, out_ref):
    ids = ids_ref[0]
    onehotT = (lax.broadcasted_iota(jnp.int32, (NUM_ELEMENTS, NB), 0)
               == ids).astype(jnp.float32)
    fused = lax.dot_general(emb_ref[...], w_ref[...],
                            (((1,), (1,)), ((), ())),
                            preferred_element_type=jnp.float32)
    out_ref[...] = lax.dot_general(onehotT, fused, (((0,), (0,)), ((), ())),
                                   preferred_element_type=jnp.float32) \
        + b_ref[...]


def _tc_nodes(ids3, embed_table, W_lin, b_row):
    grid = N_NODES // NB
    return pl.pallas_call(
        _tc_node_body,
        grid=(grid,),
        in_specs=[
            pl.BlockSpec((1, 1, NB), lambda i: (i, 0, 0)),
            pl.BlockSpec((NUM_ELEMENTS, EMBED_DIM), lambda i: (0, 0)),
            pl.BlockSpec((NODE_DIM, EMBED_DIM), lambda i: (0, 0)),
            pl.BlockSpec((1, NODE_DIM), lambda i: (0, 0)),
        ],
        out_specs=pl.BlockSpec((NB, NODE_DIM), lambda i: (i, 0)),
        out_shape=jax.ShapeDtypeStruct((N_NODES, NODE_DIM), jnp.float32),
    )(ids3, embed_table, W_lin, b_row)


def kernel(at_no, pos, edge_index, shifts, embed_table, W_lin, b_lin):
    pos_flat = pos.reshape(-1)
    src = jnp.pad(edge_index[0], (0, PAD))
    dst = jnp.pad(edge_index[1], (0, PAD))
    shT = jnp.pad(shifts.T, ((0, 0), (0, PAD)))

    vec_flat = _sc_edge_vec()(pos_flat, src, dst, shT[0], shT[1], shT[2])
    vecT = vec_flat.reshape(3, PLANE)

    rbf, fcut, rsh = _tc_edges(vecT)
    x_scalar = _tc_nodes(at_no.reshape(N_NODES // NB, 1, NB),
                         embed_table, W_lin, b_lin.reshape(1, NODE_DIM))
    return x_scalar, rbf, fcut, rsh
